# baseline (speedup 1.0000x reference)
import jax
import jax.numpy as jnp
from jax.experimental import pallas as pl
from jax.experimental.pallas import tpu as pltpu

_B, _C, _H, _W = 8, 768, 56, 56
_G = 4
_g = _C // _G
_r = 16
_HID = _g // _r
_HW = _H * _W


def _body(x_ref, w1_ref, w2_ref, o_ref):
    xb = x_ref[0]
    pooled = jnp.sum(xb, axis=1, keepdims=True).T * (1.0 / _HW)
    h = jax.nn.relu(
        jnp.dot(pooled, w1_ref[...], preferred_element_type=jnp.float32))
    att = jax.nn.sigmoid(
        jnp.dot(h, w2_ref[...], preferred_element_type=jnp.float32))
    o_ref[0] = xb * att.T


def kernel(x, W1, W2):
    W1f = jnp.zeros((_C, _G * _HID), jnp.float32)
    W2f = jnp.zeros((_G * _HID, _C), jnp.float32)
    for i in range(_G):
        W1f = W1f.at[i::_G, i * _HID:(i + 1) * _HID].set(W1[i])
        W2f = W2f.at[i * _HID:(i + 1) * _HID, i::_G].set(W2[i])

    x3 = x.reshape(_B, _C, _HW)
    out = pl.pallas_call(
        _body,
        grid=(_B,),
        in_specs=[
            pl.BlockSpec((1, _C, _HW), lambda b: (b, 0, 0)),
            pl.BlockSpec((_C, _G * _HID), lambda b: (0, 0)),
            pl.BlockSpec((_G * _HID, _C), lambda b: (0, 0)),
        ],
        out_specs=pl.BlockSpec((1, _C, _HW), lambda b: (b, 0, 0)),
        out_shape=jax.ShapeDtypeStruct((_B, _C, _HW), jnp.float32),
        compiler_params=pltpu.CompilerParams(
            dimension_semantics=("arbitrary",),
        ),
    )(x3, W1f, W2f)
    return out.reshape(_B, _C, _H, _W)
